# baseline (speedup 1.0000x reference)
import functools

import jax
import jax.numpy as jnp
from jax import lax
from jax.experimental import pallas as pl
from jax.experimental.pallas import tpu as pltpu
from jax.experimental.pallas import tpu_sc as plsc

N = 262144
D = 128

R = 4096
NB = N // R


def _tc_body(x_ref, out_ref, mask_ref, mean_ref, acc_ref, cnt_ref):
    i = pl.program_id(0)
    x = x_ref[...]
    out_ref[...] = x
    nzrow = jnp.any(x != 0.0, axis=1)
    mask_ref[0, 0, :] = jnp.where(nzrow, 0, 1).astype(jnp.int32)
    bsum = jnp.sum(x, axis=0)
    bcnt = jnp.sum(nzrow.astype(jnp.float32))

    @pl.when(i == 0)
    def _():
        acc_ref[...] = jnp.zeros_like(acc_ref)
        cnt_ref[0] = 0.0

    acc_ref[0, :] += bsum
    cnt_ref[0] += bcnt

    @pl.when(i == NB - 1)
    def _():
        mean_ref[0, :] = acc_ref[0, :] / cnt_ref[0]


_tc_pass = pl.pallas_call(
    _tc_body,
    grid=(NB,),
    in_specs=[pl.BlockSpec((R, D), lambda i: (i, 0))],
    out_specs=[
        pl.BlockSpec((R, D), lambda i: (i, 0)),
        pl.BlockSpec((1, 1, R), lambda i: (i, 0, 0)),
        pl.BlockSpec((1, D), lambda i: (0, 0)),
    ],
    out_shape=[
        jax.ShapeDtypeStruct((N, D), jnp.float32),
        jax.ShapeDtypeStruct((NB, 1, R), jnp.int32),
        jax.ShapeDtypeStruct((1, D), jnp.float32),
    ],
    scratch_shapes=[
        pltpu.VMEM((1, D), jnp.float32),
        pltpu.SMEM((1,), jnp.float32),
    ],
    compiler_params=pltpu.CompilerParams(
        dimension_semantics=("arbitrary",),
    ),
)

NC = 2
NS = 16
NW = NC * NS
ROWS_W = N // NW
L = 16
GROUPS = ROWS_W // L
CH = 128
def _sc_body(out_hbm, mask_hbm, mean_hbm, mask_v, idx2d, mean_v, rows_v, sem):
    c = lax.axis_index("c")
    s = lax.axis_index("s")
    wid = s * NC + c
    base = wid * ROWS_W
    iota = lax.iota(jnp.int32, L)

    pltpu.sync_copy(mask_hbm.at[pl.ds(base * 1, ROWS_W)], mask_v)

    pltpu.sync_copy(mean_hbm, mean_v)
    mv = [mean_v[pl.ds(j * L, L)] for j in range(D // L)]

    def repl(r, _):
        for j in range(D // L):
            rows_v[r, pl.ds(j * L, L)] = mv[j]
        return 0

    lax.fori_loop(0, CH, repl, 0)

    UNROLL = 8

    def step(j, cnt_s):
        for u in range(UNROLL):
            g = j * UNROLL + u
            m32 = mask_v[pl.ds(g * L, L)]
            mb = m32 != 0
            vidx = base + g * L + iota
            pos = cnt_s + plsc.cumsum(m32) - m32
            plsc.store_scatter(idx2d, [pos >> 7, pos & 127], vidx, mask=mb)
            cnt_s = cnt_s + plsc.all_reduce_population_count(mb)
        return cnt_s

    cnt_s = lax.fori_loop(
        0, GROUPS // UNROLL, step, jnp.zeros((L,), jnp.int32)
    )
    cnt = jnp.max(cnt_s)

    @pl.when(cnt > 0)
    def _():
        nch = (cnt + CH - 1) // CH
        tail = (nch - 1) * CH

        for j in range(CH // L):
            pos = tail + j * L + iota
            cpos = jnp.minimum(pos, cnt - 1)
            v = plsc.load_gather(idx2d, [cpos >> 7, cpos & 127])
            plsc.store_scatter(idx2d, [pos >> 7, pos & 127], v)

        def issue(ci, _):
            pltpu.async_copy(rows_v, out_hbm.at[idx2d.at[ci]], sem)
            return 0

        lax.fori_loop(0, nch, issue, 0)

        def drain(ci, _):
            pltpu.make_async_copy(rows_v, out_hbm.at[idx2d.at[0]], sem).wait()
            return 0

        lax.fori_loop(0, nch, drain, 0)


CPN = N
CPW = CPN // NW
CHR = 256
NCHK = CPW // CHR


def _sc_copy_body(x_hbm, lo_hbm, b0, b1, si0, si1, so0, so1):
    c = lax.axis_index("c")
    s = lax.axis_index("s")
    wid = s * NC + c
    row0 = wid * CPW
    bufs = (b0, b1)
    isems = (si0, si1)
    osems = (so0, so1)
    dins = [None] * NCHK
    douts = [None] * NCHK
    dins[0] = pltpu.async_copy(x_hbm.at[pl.ds(row0, CHR)], bufs[0], isems[0])
    for k in range(NCHK):
        b = k % 2
        if k + 1 < NCHK:
            if k >= 1:
                douts[k - 1].wait()
            dins[k + 1] = pltpu.async_copy(
                x_hbm.at[pl.ds(row0 + (k + 1) * CHR, CHR)],
                bufs[1 - b],
                isems[1 - b],
            )
        dins[k].wait()
        douts[k] = pltpu.async_copy(
            bufs[b], lo_hbm.at[pl.ds(row0 + k * CHR, CHR)], osems[b]
        )
    douts[NCHK - 1].wait()
    if NCHK >= 2:
        douts[NCHK - 2].wait()


@functools.cache
def _make_sc_copy():
    mesh = plsc.VectorSubcoreMesh(
        core_axis_name="c", subcore_axis_name="s", num_cores=NC, num_subcores=NS
    )
    return pl.kernel(
        _sc_copy_body,
        out_type=jax.ShapeDtypeStruct((CPN, D), jnp.float32),
        mesh=mesh,
        scratch_types=[
            pltpu.VMEM((CHR, D), jnp.float32),
            pltpu.VMEM((CHR, D), jnp.float32),
            pltpu.SemaphoreType.DMA,
            pltpu.SemaphoreType.DMA,
            pltpu.SemaphoreType.DMA,
            pltpu.SemaphoreType.DMA,
        ],
        compiler_params=pltpu.CompilerParams(
            needs_layout_passes=False, has_side_effects=True
        ),
    )


@functools.cache
def _make_sc_fill():
    mesh = plsc.VectorSubcoreMesh(
        core_axis_name="c", subcore_axis_name="s", num_cores=NC, num_subcores=NS
    )
    return pl.kernel(
        _sc_body,
        out_type=(),
        mesh=mesh,
        scratch_types=[
            pltpu.VMEM((ROWS_W,), jnp.int32),
            pltpu.VMEM((ROWS_W // CH, CH), jnp.int32),
            pltpu.VMEM((D,), jnp.float32),
            pltpu.VMEM((CH, D), jnp.float32),
            pltpu.SemaphoreType.DMA,
        ],
        compiler_params=pltpu.CompilerParams(needs_layout_passes=False),
    )


def kernel(inputs):
    _make_sc_copy()(inputs)
    tmp, mask3, mean = _tc_pass(inputs)
    mask = mask3.reshape(N)
    ref = jax.new_ref(tmp)
    _make_sc_fill()(ref, mask, mean.reshape(D))
    return jax.freeze(ref)


# speedup vs baseline: 1.4016x; 1.4016x over previous
import functools

import jax
import jax.numpy as jnp
from jax import lax
from jax.experimental import pallas as pl
from jax.experimental.pallas import tpu as pltpu
from jax.experimental.pallas import tpu_sc as plsc

N = 262144
D = 128

NC = 2
NS = 16
NW = NC * NS
L = 16
ROWS_W = N // NW
CHR = 256
NCHK = ROWS_W // CHR
CH = 128
NIDX = ROWS_W // CH


def _mesh():
    return plsc.VectorSubcoreMesh(
        core_axis_name="c", subcore_axis_name="s", num_cores=NC, num_subcores=NS
    )


def _sc_main_body(
    x_hbm, out_hbm, idx_hbm, psum_hbm, pcnt_hbm,
    b0, b1, idx2d, sumv, cntv, si0, si1, so0, so1, ss
):
    c = lax.axis_index("c")
    s = lax.axis_index("s")
    wid = s * NC + c
    row0 = wid * ROWS_W
    iota = lax.iota(jnp.int32, L)
    bufs = (b0, b1)
    isems = (si0, si1)
    osems = (so0, so1)

    pltpu.async_copy(x_hbm.at[pl.ds(row0, CHR)], b0, si0)
    pltpu.async_copy(x_hbm.at[pl.ds(row0 + CHR, CHR)], b1, si1)

    zf = jnp.zeros((L,), jnp.float32)
    zi = jnp.zeros((L,), jnp.int32)

    def chunk_compute(buf, kbase, carry):
        def row_step(r, carry):
            (a0, a1, a2, a3, a4, a5, a6, a7, cnt) = carry
            v = [buf[r, pl.ds(j * L, L)] for j in range(D // L)]
            nz = (v[0] != 0.0) | (v[1] != 0.0) | (v[2] != 0.0) | (v[3] != 0.0) \
                | (v[4] != 0.0) | (v[5] != 0.0) | (v[6] != 0.0) | (v[7] != 0.0)
            pc = plsc.all_reduce_population_count(nz)
            zero = pc == 0
            ridx = jnp.full((L,), row0 + kbase + r, jnp.int32)
            m1 = zero & (iota == 0)
            plsc.store_scatter(idx2d, [cnt >> 7, cnt & 127], ridx, mask=m1)
            cnt = cnt + jnp.where(zero, 1, 0).astype(jnp.int32)
            return (a0 + v[0], a1 + v[1], a2 + v[2], a3 + v[3],
                    a4 + v[4], a5 + v[5], a6 + v[6], a7 + v[7], cnt)

        return lax.fori_loop(0, CHR, row_step, carry)

    def body(t, carry):
        for sub in range(2):
            k = 2 * t + sub
            buf = bufs[sub]
            isem = isems[sub]
            osem = osems[sub]
            src = x_hbm.at[pl.ds(row0 + k * CHR, CHR)]
            dst = out_hbm.at[pl.ds(row0 + k * CHR, CHR)]
            pltpu.make_async_copy(src, buf, isem).wait()
            pltpu.async_copy(buf, dst, osem)
            carry = chunk_compute(buf, k * CHR, carry)
            pltpu.make_async_copy(buf, dst, osem).wait()

            @pl.when(k + 2 < NCHK)
            def _():
                pltpu.async_copy(
                    x_hbm.at[pl.ds(row0 + (k + 2) * CHR, CHR)], buf, isem
                )

        return carry

    init = (zf, zf, zf, zf, zf, zf, zf, zf, zi)
    res = lax.fori_loop(0, NCHK // 2, body, init)

    for j in range(D // L):
        sumv[pl.ds(j * L, L)] = res[j]
    cntv[pl.ds(0, L)] = res[8]
    pltpu.sync_copy(sumv, psum_hbm.at[wid])
    pltpu.sync_copy(cntv, pcnt_hbm.at[wid])
    pltpu.sync_copy(idx2d, idx_hbm.at[wid])


@functools.cache
def _make_sc_main():
    return pl.kernel(
        _sc_main_body,
        out_type=(
            jax.ShapeDtypeStruct((N, D), jnp.float32),
            jax.ShapeDtypeStruct((NW, NIDX, CH), jnp.int32),
            jax.ShapeDtypeStruct((NW, D), jnp.float32),
            jax.ShapeDtypeStruct((NW, L), jnp.int32),
        ),
        mesh=_mesh(),
        scratch_types=[
            pltpu.VMEM((CHR, D), jnp.float32),
            pltpu.VMEM((CHR, D), jnp.float32),
            pltpu.VMEM((NIDX, CH), jnp.int32),
            pltpu.VMEM((D,), jnp.float32),
            pltpu.VMEM((L,), jnp.int32),
            pltpu.SemaphoreType.DMA,
            pltpu.SemaphoreType.DMA,
            pltpu.SemaphoreType.DMA,
            pltpu.SemaphoreType.DMA,
            pltpu.SemaphoreType.DMA,
        ],
        compiler_params=pltpu.CompilerParams(needs_layout_passes=False),
    )


def _sc_fill_body(
    out_hbm, idx_hbm, psum_hbm, pcnt_hbm, psv, pcv, idx2d, rows_v, sem
):
    c = lax.axis_index("c")
    s = lax.axis_index("s")
    wid = s * NC + c
    iota = lax.iota(jnp.int32, L)

    pltpu.sync_copy(psum_hbm, psv)
    pltpu.sync_copy(pcnt_hbm, pcv)
    pltpu.sync_copy(idx_hbm.at[wid], idx2d)

    tot = jnp.zeros((L,), jnp.int32)
    for w in range(NW):
        tot = tot + pcv[w, pl.ds(0, L)]
    cntf = (jnp.full((L,), N, jnp.int32) - tot).astype(jnp.float32)

    mv = []
    for j in range(D // L):
        acc = jnp.zeros((L,), jnp.float32)
        for w in range(NW):
            acc = acc + psv[w, pl.ds(j * L, L)]
        mv.append(acc / cntf)

    def repl(r, _):
        for j in range(D // L):
            rows_v[r, pl.ds(j * L, L)] = mv[j]
        return 0

    lax.fori_loop(0, CH, repl, 0)

    cnt = jnp.max(pcv[wid, pl.ds(0, L)])

    @pl.when(cnt > 0)
    def _():
        nch = (cnt + CH - 1) // CH
        tail = (nch - 1) * CH

        for j in range(CH // L):
            pos = tail + j * L + iota
            cpos = jnp.minimum(pos, cnt - 1)
            v = plsc.load_gather(idx2d, [cpos >> 7, cpos & 127])
            plsc.store_scatter(idx2d, [pos >> 7, pos & 127], v)

        def issue(ci, _):
            pltpu.async_copy(rows_v, out_hbm.at[idx2d.at[ci]], sem)
            return 0

        lax.fori_loop(0, nch, issue, 0)

        def drain(ci, _):
            pltpu.make_async_copy(rows_v, out_hbm.at[idx2d.at[0]], sem).wait()
            return 0

        lax.fori_loop(0, nch, drain, 0)


@functools.cache
def _make_sc_fill():
    return pl.kernel(
        _sc_fill_body,
        out_type=(),
        mesh=_mesh(),
        scratch_types=[
            pltpu.VMEM((NW, D), jnp.float32),
            pltpu.VMEM((NW, L), jnp.int32),
            pltpu.VMEM((NIDX, CH), jnp.int32),
            pltpu.VMEM((CH, D), jnp.float32),
            pltpu.SemaphoreType.DMA,
        ],
        compiler_params=pltpu.CompilerParams(needs_layout_passes=False),
    )


def kernel(inputs):
    out, idx, psum, pcnt = _make_sc_main()(inputs)
    ref = jax.new_ref(out)
    _make_sc_fill()(ref, idx, psum, pcnt)
    return jax.freeze(ref)
